# baseline (speedup 1.0000x reference)
import functools

import jax
import jax.numpy as jnp
from jax.experimental import pallas as pl
from jax.experimental.pallas import tpu as pltpu
from jax.experimental.pallas import tpu_sc as plsc

K = 20
C = 256
W = 2
L = 16


def _sc_body(f_v, x_v, o_v):
    for r in range(W):
        for ci in range(C // L):
            c0 = ci * L
            cv = f_v[0, r, pl.ds(c0, L)]
            for k in range(K):
                xv = x_v[k, r, pl.ds(c0, L)]
                o_v[k, r, pl.ds(c0, L)] = cv
                o_v[k, r, pl.ds(C + c0, L)] = xv - cv


def kernel(fts, knn_fts):
    B, N, _ = fts.shape
    knn_t = jnp.transpose(knn_fts, (0, 2, 1, 3))
    x3 = knn_t.reshape(B * K, N, C)

    mesh = plsc.VectorSubcoreMesh(core_axis_name="c", subcore_axis_name="s")

    @functools.partial(
        pl.kernel,
        out_type=jax.ShapeDtypeStruct((B * K, N, 2 * C), fts.dtype),
        mesh=mesh,
    )
    def sck(f_hbm, x_hbm, o_hbm):
        pltpu.emit_pipeline(
            _sc_body,
            grid=(B, N // W),
            in_specs=[
                pl.BlockSpec((1, W, C), lambda b, j: (b, j, 0)),
                pl.BlockSpec((K, W, C), lambda b, j: (b, j, 0)),
            ],
            out_specs=[pl.BlockSpec((K, W, 2 * C), lambda b, j: (b, j, 0))],
            core_axis_name=("c", "s"),
            dimension_semantics=(pltpu.PARALLEL, pltpu.PARALLEL),
        )(f_hbm, x_hbm, o_hbm)

    out3 = sck(fts, x3)
    out_t = out3.reshape(B, K, N, 2 * C)
    return jnp.transpose(out_t, (0, 2, 1, 3))


# speedup vs baseline: 1.2430x; 1.2430x over previous
import functools

import jax
import jax.numpy as jnp
from jax.experimental import pallas as pl
from jax.experimental.pallas import tpu as pltpu
from jax.experimental.pallas import tpu_sc as plsc

K = 20
C = 256
W = 32
L = 16


def _sc_body(f_v, x_v, o_v):
    @pl.loop(0, W)
    def _(r):
        for ci in range(C // L):
            c0 = ci * L
            cv = f_v[0, r, pl.ds(c0, L)]
            xv = x_v[0, r, pl.ds(c0, L)]
            o_v[0, r, pl.ds(c0, L)] = cv
            o_v[0, r, pl.ds(C + c0, L)] = xv - cv


def kernel(fts, knn_fts):
    B, N, _ = fts.shape
    knn_t = jnp.transpose(knn_fts, (0, 2, 1, 3))
    x3 = knn_t.reshape(B * K, N, C)

    mesh = plsc.VectorSubcoreMesh(core_axis_name="c", subcore_axis_name="s")

    @functools.partial(
        pl.kernel,
        out_type=jax.ShapeDtypeStruct((B * K, N, 2 * C), fts.dtype),
        mesh=mesh,
    )
    def sck(f_hbm, x_hbm, o_hbm):
        pltpu.emit_pipeline(
            _sc_body,
            grid=(B * K, N // W),
            in_specs=[
                pl.BlockSpec((1, W, C), lambda s, j: (s // K, j, 0)),
                pl.BlockSpec((1, W, C), lambda s, j: (s, j, 0)),
            ],
            out_specs=[pl.BlockSpec((1, W, 2 * C), lambda s, j: (s, j, 0))],
            core_axis_name=("c", "s"),
            dimension_semantics=(pltpu.PARALLEL, pltpu.PARALLEL),
        )(f_hbm, x_hbm, o_hbm)

    out3 = sck(fts, x3)
    out_t = out3.reshape(B, K, N, 2 * C)
    return jnp.transpose(out_t, (0, 2, 1, 3))


# speedup vs baseline: 3.5825x; 2.8822x over previous
import functools

import jax
import jax.numpy as jnp
from jax.experimental import pallas as pl
from jax.experimental.pallas import tpu as pltpu

K = 20
C = 256
NBUF = 4


def _body(fts_hbm, x_hbm, o_hbm, c_buf, x_buf, o_buf, csem, xsem, osem,
          *, nchunk):
    s = pl.program_id(0)
    slot = jax.lax.rem(s, NBUF)

    def in_copy(j):
        return pltpu.make_async_copy(
            x_hbm.at[j], x_buf.at[jax.lax.rem(j, NBUF)],
            xsem.at[jax.lax.rem(j, NBUF)])

    def out_copy(j):
        return pltpu.make_async_copy(
            o_buf.at[jax.lax.rem(j, NBUF)], o_hbm.at[j],
            osem.at[jax.lax.rem(j, NBUF)])

    @pl.when(s == 0)
    def _prologue():
        for j in range(min(NBUF, nchunk)):
            in_copy(j).start()

    @pl.when(jax.lax.rem(s, K) == 0)
    def _center():
        pltpu.make_async_copy(fts_hbm.at[s // K], c_buf, csem).start()
        pltpu.make_async_copy(fts_hbm.at[s // K], c_buf, csem).wait()

    in_copy(s).wait()

    @pl.when(s >= NBUF)
    def _wait_out():
        out_copy(s - NBUF).wait()

    c = c_buf[...]
    o_buf[slot, :, :C] = c
    o_buf[slot, :, C:] = x_buf[slot] - c

    out_copy(s).start()

    @pl.when(s + NBUF < nchunk)
    def _prefetch():
        in_copy(s + NBUF).start()

    @pl.when(s == nchunk - 1)
    def _epilogue():
        for j in range(max(nchunk - NBUF, 0), nchunk):
            out_copy(j).wait()


def kernel(fts, knn_fts):
    B, N, _ = fts.shape
    knn_t = jnp.transpose(knn_fts, (0, 2, 1, 3))
    x3 = knn_t.reshape(B * K, N, C)
    nchunk = B * K
    out3 = pl.pallas_call(
        functools.partial(_body, nchunk=nchunk),
        grid=(nchunk,),
        in_specs=[
            pl.BlockSpec(memory_space=pltpu.HBM),
            pl.BlockSpec(memory_space=pltpu.HBM),
        ],
        out_specs=pl.BlockSpec(memory_space=pltpu.HBM),
        out_shape=jax.ShapeDtypeStruct((B * K, N, 2 * C), fts.dtype),
        scratch_shapes=[
            pltpu.VMEM((N, C), fts.dtype),
            pltpu.VMEM((NBUF, N, C), fts.dtype),
            pltpu.VMEM((NBUF, N, 2 * C), fts.dtype),
            pltpu.SemaphoreType.DMA,
            pltpu.SemaphoreType.DMA((NBUF,)),
            pltpu.SemaphoreType.DMA((NBUF,)),
        ],
    )(fts, x3)
    out_t = out3.reshape(B, K, N, 2 * C)
    return jnp.transpose(out_t, (0, 2, 1, 3))


# speedup vs baseline: 3.7377x; 1.0433x over previous
import functools

import jax
import jax.numpy as jnp
from jax.experimental import pallas as pl
from jax.experimental.pallas import tpu as pltpu

K = 20
C = 256
NBUF = 6


def _body(fts_hbm, x_hbm, o_hbm, c_buf, x_buf, o_buf, csem, cvsem, xsem, osem,
          *, nchunk):
    s = pl.program_id(0)
    slot = jax.lax.rem(s, NBUF)

    def in_copy(j):
        return pltpu.make_async_copy(
            x_hbm.at[j], x_buf.at[jax.lax.rem(j, NBUF)],
            xsem.at[jax.lax.rem(j, NBUF)])

    def out_copy(j):
        return pltpu.make_async_copy(
            o_buf.at[jax.lax.rem(j, NBUF)], o_hbm.at[j],
            osem.at[jax.lax.rem(j, NBUF)])

    @pl.when(s == 0)
    def _prologue():
        for j in range(min(NBUF, nchunk)):
            in_copy(j).start()

    @pl.when(jax.lax.rem(s, K) == 0)
    def _center():
        pltpu.make_async_copy(fts_hbm.at[s // K], c_buf, csem).start()
        pltpu.make_async_copy(fts_hbm.at[s // K], c_buf, csem).wait()

    in_copy(s).wait()

    @pl.when(s >= NBUF)
    def _wait_out():
        out_copy(s - NBUF).wait()

    ccopy = pltpu.make_async_copy(c_buf, o_buf.at[slot, :, pl.ds(0, C)], cvsem)
    ccopy.start()
    o_buf[slot, :, C:] = x_buf[slot] - c_buf[...]
    ccopy.wait()

    out_copy(s).start()

    @pl.when(s + NBUF < nchunk)
    def _prefetch():
        in_copy(s + NBUF).start()

    @pl.when(s == nchunk - 1)
    def _epilogue():
        for j in range(max(nchunk - NBUF, 0), nchunk):
            out_copy(j).wait()


def kernel(fts, knn_fts):
    B, N, _ = fts.shape
    knn_t = jnp.transpose(knn_fts, (0, 2, 1, 3))
    x3 = knn_t.reshape(B * K, N, C)
    nchunk = B * K
    out3 = pl.pallas_call(
        functools.partial(_body, nchunk=nchunk),
        grid=(nchunk,),
        in_specs=[
            pl.BlockSpec(memory_space=pltpu.HBM),
            pl.BlockSpec(memory_space=pltpu.HBM),
        ],
        out_specs=pl.BlockSpec(memory_space=pltpu.HBM),
        out_shape=jax.ShapeDtypeStruct((B * K, N, 2 * C), fts.dtype),
        scratch_shapes=[
            pltpu.VMEM((N, C), fts.dtype),
            pltpu.VMEM((NBUF, N, C), fts.dtype),
            pltpu.VMEM((NBUF, N, 2 * C), fts.dtype),
            pltpu.SemaphoreType.DMA,
            pltpu.SemaphoreType.DMA,
            pltpu.SemaphoreType.DMA((NBUF,)),
            pltpu.SemaphoreType.DMA((NBUF,)),
        ],
    )(fts, x3)
    out_t = out3.reshape(B, K, N, 2 * C)
    return jnp.transpose(out_t, (0, 2, 1, 3))


# speedup vs baseline: 3.7438x; 1.0016x over previous
import functools

import jax
import jax.numpy as jnp
from jax.experimental import pallas as pl
from jax.experimental.pallas import tpu as pltpu

K = 20
C = 256
NBUF = 6


def _body(fts_hbm, x_hbm, o_hbm, c_buf, x_buf, o_buf, csem, xsem, osem,
          *, nchunk):
    s = pl.program_id(0)
    slot = jax.lax.rem(s, NBUF)

    def in_copy(j):
        return pltpu.make_async_copy(
            x_hbm.at[j], x_buf.at[jax.lax.rem(j, NBUF)],
            xsem.at[jax.lax.rem(j, NBUF)])

    def out_copy_center(j):
        return pltpu.make_async_copy(
            c_buf.at[jax.lax.rem(j // K, 2)],
            o_hbm.at[j, :, pl.ds(0, C)],
            osem.at[jax.lax.rem(j, NBUF)])

    def out_copy_sub(j):
        return pltpu.make_async_copy(
            o_buf.at[jax.lax.rem(j, NBUF)],
            o_hbm.at[j, :, pl.ds(C, C)],
            osem.at[jax.lax.rem(j, NBUF)])

    @pl.when(s == 0)
    def _prologue():
        for j in range(min(NBUF, nchunk)):
            in_copy(j).start()

    @pl.when(jax.lax.rem(s, K) == 0)
    def _center():
        cb = jax.lax.rem(s // K, 2)
        pltpu.make_async_copy(fts_hbm.at[s // K], c_buf.at[cb], csem).start()
        pltpu.make_async_copy(fts_hbm.at[s // K], c_buf.at[cb], csem).wait()

    in_copy(s).wait()

    @pl.when(s >= NBUF)
    def _wait_out():
        out_copy_center(s - NBUF).wait()
        out_copy_sub(s - NBUF).wait()

    out_copy_center(s).start()
    o_buf[slot] = x_buf[slot] - c_buf[jax.lax.rem(s // K, 2)]
    out_copy_sub(s).start()

    @pl.when(s + NBUF < nchunk)
    def _prefetch():
        in_copy(s + NBUF).start()

    @pl.when(s == nchunk - 1)
    def _epilogue():
        for j in range(max(nchunk - NBUF, 0), nchunk):
            out_copy_center(j).wait()
            out_copy_sub(j).wait()


def kernel(fts, knn_fts):
    B, N, _ = fts.shape
    knn_t = jnp.transpose(knn_fts, (0, 2, 1, 3))
    x3 = knn_t.reshape(B * K, N, C)
    nchunk = B * K
    out3 = pl.pallas_call(
        functools.partial(_body, nchunk=nchunk),
        grid=(nchunk,),
        in_specs=[
            pl.BlockSpec(memory_space=pltpu.HBM),
            pl.BlockSpec(memory_space=pltpu.HBM),
        ],
        out_specs=pl.BlockSpec(memory_space=pltpu.HBM),
        out_shape=jax.ShapeDtypeStruct((B * K, N, 2 * C), fts.dtype),
        scratch_shapes=[
            pltpu.VMEM((2, N, C), fts.dtype),
            pltpu.VMEM((NBUF, N, C), fts.dtype),
            pltpu.VMEM((NBUF, N, C), fts.dtype),
            pltpu.SemaphoreType.DMA,
            pltpu.SemaphoreType.DMA((NBUF,)),
            pltpu.SemaphoreType.DMA((NBUF,)),
        ],
    )(fts, x3)
    out_t = out3.reshape(B, K, N, 2 * C)
    return jnp.transpose(out_t, (0, 2, 1, 3))


# speedup vs baseline: 3.7732x; 1.0078x over previous
import functools

import jax
import jax.numpy as jnp
from jax.experimental import pallas as pl
from jax.experimental.pallas import tpu as pltpu

K = 20
C = 256
NBUF = 8


def _body(fts_hbm, x_hbm, o_hbm, c_buf, x_buf, o_buf, csem, xsem, osem,
          *, nchunk):
    s = pl.program_id(0)
    slot = jax.lax.rem(s, NBUF)

    def in_copy(j):
        return pltpu.make_async_copy(
            x_hbm.at[j], x_buf.at[jax.lax.rem(j, NBUF)],
            xsem.at[jax.lax.rem(j, NBUF)])

    def out_copy_center(j):
        return pltpu.make_async_copy(
            c_buf.at[jax.lax.rem(j // K, 2)],
            o_hbm.at[j, :, pl.ds(0, C)],
            osem.at[jax.lax.rem(j, NBUF)])

    def out_copy_sub(j):
        return pltpu.make_async_copy(
            o_buf.at[jax.lax.rem(j, NBUF)],
            o_hbm.at[j, :, pl.ds(C, C)],
            osem.at[jax.lax.rem(j, NBUF)])

    @pl.when(s == 0)
    def _prologue():
        for j in range(min(NBUF, nchunk)):
            in_copy(j).start()

    @pl.when(jax.lax.rem(s, K) == 0)
    def _center():
        cb = jax.lax.rem(s // K, 2)
        pltpu.make_async_copy(fts_hbm.at[s // K], c_buf.at[cb], csem).start()
        pltpu.make_async_copy(fts_hbm.at[s // K], c_buf.at[cb], csem).wait()

    in_copy(s).wait()

    @pl.when(s >= NBUF)
    def _wait_out():
        out_copy_center(s - NBUF).wait()
        out_copy_sub(s - NBUF).wait()

    out_copy_center(s).start()
    o_buf[slot] = x_buf[slot] - c_buf[jax.lax.rem(s // K, 2)]
    out_copy_sub(s).start()

    @pl.when(s + NBUF < nchunk)
    def _prefetch():
        in_copy(s + NBUF).start()

    @pl.when(s == nchunk - 1)
    def _epilogue():
        for j in range(max(nchunk - NBUF, 0), nchunk):
            out_copy_center(j).wait()
            out_copy_sub(j).wait()


def kernel(fts, knn_fts):
    B, N, _ = fts.shape
    knn_t = jnp.transpose(knn_fts, (0, 2, 1, 3))
    x3 = knn_t.reshape(B * K, N, C)
    nchunk = B * K
    out3 = pl.pallas_call(
        functools.partial(_body, nchunk=nchunk),
        grid=(nchunk,),
        in_specs=[
            pl.BlockSpec(memory_space=pltpu.HBM),
            pl.BlockSpec(memory_space=pltpu.HBM),
        ],
        out_specs=pl.BlockSpec(memory_space=pltpu.HBM),
        out_shape=jax.ShapeDtypeStruct((B * K, N, 2 * C), fts.dtype),
        scratch_shapes=[
            pltpu.VMEM((2, N, C), fts.dtype),
            pltpu.VMEM((NBUF, N, C), fts.dtype),
            pltpu.VMEM((NBUF, N, C), fts.dtype),
            pltpu.SemaphoreType.DMA,
            pltpu.SemaphoreType.DMA((NBUF,)),
            pltpu.SemaphoreType.DMA((NBUF,)),
        ],
    )(fts, x3)
    out_t = out3.reshape(B, K, N, 2 * C)
    return jnp.transpose(out_t, (0, 2, 1, 3))
